# baseline (speedup 1.0000x reference)
import functools

import numpy as np
import jax
import jax.numpy as jnp
from jax import lax
from jax.experimental import pallas as pl
from jax.experimental.pallas import tpu as pltpu
from jax.experimental.pallas import tpu_sc as plsc

_B, _S, _W, _D = 8, 16, 32, 64
_ND = _S * _W
_NV = 4096
_NODE = 32
_VT = 1024
_NVT = _NV // _VT
_EDGES = [float(x) for x in np.linspace(-0.5, 0.99, 15)]


_NW = 32
_RPW = (_B * _ND) // _NW
_VOCAB = 100001
_SPLIT = 50176
_TBLK = 391


def _repack_table(table_t):
  def body(a_ref, b_ref, out_ref):
    out_ref[:, 0:_D] = jnp.transpose(a_ref[...])
    out_ref[:, _D:128] = jnp.transpose(b_ref[...])

  nblk = _SPLIT // 512
  return pl.pallas_call(
      body,
      grid=(nblk,),
      in_specs=[
          pl.BlockSpec((_D, 512), lambda c: (0, c)),
          pl.BlockSpec((_D, 512), lambda c: (0, c + nblk)),
      ],
      out_specs=pl.BlockSpec((512, 128), lambda c: (c, 0)),
      out_shape=jax.ShapeDtypeStruct((_SPLIT, 128), jnp.float32),
  )(table_t, table_t)


def _sc_gather(table, idx):
  mesh = plsc.VectorSubcoreMesh(core_axis_name="c", subcore_axis_name="s")

  @functools.partial(
      pl.kernel,
      mesh=mesh,
      compiler_params=pltpu.CompilerParams(use_tc_tiling_on_sc=False),
      out_type=jax.ShapeDtypeStruct((_B * _ND, 128), jnp.float32),
      scratch_types=[
          pltpu.VMEM((_RPW,), jnp.int32),
          pltpu.VMEM((_RPW,), jnp.int32),
          pltpu.VMEM((_RPW, _D), jnp.float32),
          pltpu.SemaphoreType.DMA,
      ],
  )
  def gather_kernel(table_hbm, idx_hbm, out_hbm, idx_v, idx2_v, rows_v, sem):
    wid = lax.axis_index("s") * 2 + lax.axis_index("c")
    base = wid * _RPW
    for i in range(4):
      pltpu.sync_copy(idx_hbm.at[wid * 4 + i, pl.ds(0, _W)],
                      idx_v.at[pl.ds(i * _W, _W)])
    for i in range(_RPW // 16):
      v = idx_v[pl.ds(i * 16, 16)]
      idx2_v[pl.ds(i * 16, 16)] = v + v + jnp.where(
          v < _SPLIT, 0, 1 - 2 * _SPLIT)
    pltpu.async_copy(table_hbm.at[idx2_v], rows_v, sem).wait()
    pltpu.sync_copy(rows_v, out_hbm.at[pl.ds(base, _RPW), pl.ds(0, _D)])

  return gather_kernel(table, idx)


def _tc_body(demb_ref, vvt_ref, imp_ref, phi_ref, ww_ref, ws_ref,
             diff_ref, start_ref, final_ref, attn_ref, attn_scr):
  vt = pl.program_id(1)

  @pl.when(vt == 0)
  def _():
    demb = demb_ref[:, :_D]
    wl = jnp.sum(demb * ww_ref[...], axis=1, keepdims=True)
    wl3 = wl.reshape(_S, _W, 1)
    wmax = jnp.max(wl3, axis=1, keepdims=True)
    we = jnp.exp(wl3 - wmax)
    wa = we / jnp.sum(we, axis=1, keepdims=True)
    sl = jnp.sum(imp_ref[...] * ws_ref[...], axis=1, keepdims=True)
    smax = jnp.max(sl, axis=0, keepdims=True)
    se = jnp.exp(sl - smax)
    sa = se / jnp.sum(se, axis=0, keepdims=True)
    attn = (wa * sa.reshape(_S, 1, 1)).reshape(_ND, 1)
    attn_scr[...] = attn
    attn_ref[...] = attn

  sim = jnp.dot(demb_ref[:, :_D], vvt_ref[...],
                preferred_element_type=jnp.float32)
  bw = [start_ref[0] + jnp.maximum(diff_ref[0], 0.0)]
  for k in range(1, 16):
    bw.append(bw[-1] + jnp.maximum(diff_ref[k], 0.0))
  b3 = sim >= _EDGES[7]
  b2 = sim >= jnp.where(b3, _EDGES[11], _EDGES[3])
  e1 = jnp.where(b3, jnp.where(b2, _EDGES[13], _EDGES[9]),
                 jnp.where(b2, _EDGES[5], _EDGES[1]))
  b1 = sim >= e1
  e0hi = jnp.where(b3, jnp.where(b2, _EDGES[14], _EDGES[10]),
                   jnp.where(b2, _EDGES[6], _EDGES[2]))
  e0lo = jnp.where(b3, jnp.where(b2, _EDGES[12], _EDGES[8]),
                   jnp.where(b2, _EDGES[4], _EDGES[0]))
  b0 = sim >= jnp.where(b1, e0hi, e0lo)
  t = [jnp.where(b0, bw[2 * i + 1], bw[2 * i]) for i in range(8)]
  u = [jnp.where(b1, t[2 * j + 1], t[2 * j]) for j in range(4)]
  p = [jnp.where(b2, u[1], u[0]), jnp.where(b2, u[3], u[2])]
  w = jnp.where(b3, p[1], p[0])
  weighted = lax.dot_general(attn_scr[...], w,
                             (((0,), (0,)), ((), ())),
                             preferred_element_type=jnp.float32)
  part = lax.dot_general(weighted, phi_ref[...],
                         (((1,), (1,)), ((), ())),
                         preferred_element_type=jnp.float32)

  @pl.when(vt == 0)
  def _():
    final_ref[...] = part

  @pl.when(vt != 0)
  def _():
    final_ref[...] = final_ref[...] + part


def _tc_call(demb3, VvT, imp, phi, ww2, ws2, diff, start):
  return pl.pallas_call(
      _tc_body,
      grid=(_B, _NVT),
      in_specs=[
          pl.BlockSpec((_ND, 128), lambda b, v: (b, 0)),
          pl.BlockSpec((_D, _VT), lambda b, v: (0, v)),
          pl.BlockSpec((None, _S, _D), lambda b, v: (b, 0, 0)),
          pl.BlockSpec((_NODE, _VT), lambda b, v: (0, v)),
          pl.BlockSpec((1, _D), lambda b, v: (0, 0)),
          pl.BlockSpec((1, _D), lambda b, v: (0, 0)),
          pl.BlockSpec(memory_space=pltpu.SMEM),
          pl.BlockSpec(memory_space=pltpu.SMEM),
      ],
      out_specs=[
          pl.BlockSpec((None, 1, _NODE), lambda b, v: (b, 0, 0)),
          pl.BlockSpec((None, _ND, 1), lambda b, v: (b, 0, 0)),
      ],
      out_shape=[
          jax.ShapeDtypeStruct((_B, 1, _NODE), jnp.float32),
          jax.ShapeDtypeStruct((_B, _ND, 1), jnp.float32),
      ],
      scratch_shapes=[pltpu.VMEM((_ND, 1), jnp.float32)],
  )(demb3, VvT, imp, phi, ww2, ws2, diff, start)


def kernel(input_ids, ImportanceFeatureMat, labels, embedding_table, VvT,
           phi_vs, bin_weight_difference, bin_weight_difference_start,
           w_word, w_sent):
  ids = jnp.pad(input_ids.astype(jnp.int32),
                ((0, 0), (0, 0), (0, 128 - _W))).reshape(_B * _S, 128)
  tbl = _repack_table(embedding_table.T).reshape(2 * _SPLIT, _D)
  demb = _sc_gather(tbl, ids)
  final, attn = _tc_call(
      demb, VvT, ImportanceFeatureMat, phi_vs,
      w_word.reshape(1, _D), w_sent.reshape(1, _D),
      bin_weight_difference, bin_weight_difference_start)
  return final.reshape(_B, _NODE), attn.reshape(_B, _ND)


# speedup vs baseline: 1.1820x; 1.1820x over previous
import functools

import numpy as np
import jax
import jax.numpy as jnp
from jax import lax
from jax.experimental import pallas as pl
from jax.experimental.pallas import tpu as pltpu
from jax.experimental.pallas import tpu_sc as plsc

_B, _S, _W, _D = 8, 16, 32, 64
_ND = _S * _W
_NV = 4096
_NODE = 32
_VT = 2048
_CH = 512
_NVT = _NV // _VT
_EDGES = [float(x) for x in np.linspace(-0.5, 0.99, 15)]


_NW = 32
_RPW = (_B * _ND) // _NW
_VOCAB = 100001
_SPLIT = 50176
_TBLK = 391


def _repack_table(table_t):
  def body(a_ref, b_ref, out_ref):
    out_ref[:, 0:_D] = jnp.transpose(a_ref[...])
    out_ref[:, _D:128] = jnp.transpose(b_ref[...])

  nblk = _SPLIT // 1024
  return pl.pallas_call(
      body,
      grid=(nblk,),
      in_specs=[
          pl.BlockSpec((_D, 1024), lambda c: (0, c)),
          pl.BlockSpec((_D, 1024), lambda c: (0, c + nblk)),
      ],
      out_specs=pl.BlockSpec((1024, 128), lambda c: (c, 0)),
      out_shape=jax.ShapeDtypeStruct((_SPLIT, 128), jnp.float32),
  )(table_t, table_t)


def _sc_gather(table, idx):
  mesh = plsc.VectorSubcoreMesh(core_axis_name="c", subcore_axis_name="s")

  @functools.partial(
      pl.kernel,
      mesh=mesh,
      compiler_params=pltpu.CompilerParams(use_tc_tiling_on_sc=False),
      out_type=jax.ShapeDtypeStruct((_B * _ND, 128), jnp.float32),
      scratch_types=[
          pltpu.VMEM((_RPW,), jnp.int32),
          pltpu.VMEM((_RPW,), jnp.int32),
          pltpu.VMEM((_RPW, _D), jnp.float32),
          pltpu.SemaphoreType.DMA,
      ],
  )
  def gather_kernel(table_hbm, idx_hbm, out_hbm, idx_v, idx2_v, rows_v, sem):
    wid = lax.axis_index("s") * 2 + lax.axis_index("c")
    base = wid * _RPW
    for i in range(4):
      pltpu.sync_copy(idx_hbm.at[wid * 4 + i, pl.ds(0, _W)],
                      idx_v.at[pl.ds(i * _W, _W)])
    for i in range(_RPW // 16):
      v = idx_v[pl.ds(i * 16, 16)]
      idx2_v[pl.ds(i * 16, 16)] = v + v + jnp.where(
          v < _SPLIT, 0, 1 - 2 * _SPLIT)
    pltpu.async_copy(table_hbm.at[idx2_v], rows_v, sem).wait()
    pltpu.sync_copy(rows_v, out_hbm.at[pl.ds(base, _RPW), pl.ds(0, _D)])

  return gather_kernel(table, idx)


def _tc_body(demb_ref, vvt_ref, imp_ref, phi_ref, ww_ref, ws_ref,
             diff_ref, start_ref, final_ref, attn_ref, attn_scr):
  vt = pl.program_id(1)

  @pl.when(vt == 0)
  def _():
    demb = demb_ref[:, :_D]
    wl = jnp.sum(demb * ww_ref[...], axis=1, keepdims=True)
    wl3 = wl.reshape(_S, _W, 1)
    wmax = jnp.max(wl3, axis=1, keepdims=True)
    we = jnp.exp(wl3 - wmax)
    wa = we / jnp.sum(we, axis=1, keepdims=True)
    sl = jnp.sum(imp_ref[...] * ws_ref[...], axis=1, keepdims=True)
    smax = jnp.max(sl, axis=0, keepdims=True)
    se = jnp.exp(sl - smax)
    sa = se / jnp.sum(se, axis=0, keepdims=True)
    attn = (wa * sa.reshape(_S, 1, 1)).reshape(_ND, 1)
    attn_scr[...] = attn
    attn_ref[...] = attn

  bw = [start_ref[0] + jnp.maximum(diff_ref[0], 0.0)]
  for k in range(1, 16):
    bw.append(bw[-1] + jnp.maximum(diff_ref[k], 0.0))
  attn = attn_scr[...]
  part = None
  for c in range(_VT // _CH):
    sim = jnp.dot(demb_ref[:, :_D], vvt_ref[:, c * _CH:(c + 1) * _CH],
                  preferred_element_type=jnp.float32)
    b3 = sim >= _EDGES[7]
    b2 = sim >= jnp.where(b3, _EDGES[11], _EDGES[3])
    e1 = jnp.where(b3, jnp.where(b2, _EDGES[13], _EDGES[9]),
                   jnp.where(b2, _EDGES[5], _EDGES[1]))
    b1 = sim >= e1
    e0hi = jnp.where(b3, jnp.where(b2, _EDGES[14], _EDGES[10]),
                     jnp.where(b2, _EDGES[6], _EDGES[2]))
    e0lo = jnp.where(b3, jnp.where(b2, _EDGES[12], _EDGES[8]),
                     jnp.where(b2, _EDGES[4], _EDGES[0]))
    b0 = sim >= jnp.where(b1, e0hi, e0lo)
    t = [jnp.where(b0, bw[2 * i + 1], bw[2 * i]) for i in range(8)]
    u = [jnp.where(b1, t[2 * j + 1], t[2 * j]) for j in range(4)]
    p = [jnp.where(b2, u[1], u[0]), jnp.where(b2, u[3], u[2])]
    w = jnp.where(b3, p[1], p[0])
    weighted = lax.dot_general(attn, w,
                               (((0,), (0,)), ((), ())),
                               preferred_element_type=jnp.float32)
    pc = lax.dot_general(weighted, phi_ref[:, c * _CH:(c + 1) * _CH],
                         (((1,), (1,)), ((), ())),
                         preferred_element_type=jnp.float32)
    part = pc if part is None else part + pc

  @pl.when(vt == 0)
  def _():
    final_ref[...] = part

  @pl.when(vt != 0)
  def _():
    final_ref[...] = final_ref[...] + part


def _tc_call(demb3, VvT, imp, phi, ww2, ws2, diff, start):
  return pl.pallas_call(
      _tc_body,
      grid=(_B, _NVT),
      in_specs=[
          pl.BlockSpec((_ND, 128), lambda b, v: (b, 0)),
          pl.BlockSpec((_D, _VT), lambda b, v: (0, v)),
          pl.BlockSpec((None, _S, _D), lambda b, v: (b, 0, 0)),
          pl.BlockSpec((_NODE, _VT), lambda b, v: (0, v)),
          pl.BlockSpec((1, _D), lambda b, v: (0, 0)),
          pl.BlockSpec((1, _D), lambda b, v: (0, 0)),
          pl.BlockSpec(memory_space=pltpu.SMEM),
          pl.BlockSpec(memory_space=pltpu.SMEM),
      ],
      out_specs=[
          pl.BlockSpec((None, 1, _NODE), lambda b, v: (b, 0, 0)),
          pl.BlockSpec((None, _ND, 1), lambda b, v: (b, 0, 0)),
      ],
      out_shape=[
          jax.ShapeDtypeStruct((_B, 1, _NODE), jnp.float32),
          jax.ShapeDtypeStruct((_B, _ND, 1), jnp.float32),
      ],
      scratch_shapes=[pltpu.VMEM((_ND, 1), jnp.float32)],
  )(demb3, VvT, imp, phi, ww2, ws2, diff, start)


def kernel(input_ids, ImportanceFeatureMat, labels, embedding_table, VvT,
           phi_vs, bin_weight_difference, bin_weight_difference_start,
           w_word, w_sent):
  ids = jnp.pad(input_ids.astype(jnp.int32),
                ((0, 0), (0, 0), (0, 128 - _W))).reshape(_B * _S, 128)
  tbl = _repack_table(embedding_table.T).reshape(2 * _SPLIT, _D)
  demb = _sc_gather(tbl, ids)
  final, attn = _tc_call(
      demb, VvT, ImportanceFeatureMat, phi_vs,
      w_word.reshape(1, _D), w_sent.reshape(1, _D),
      bin_weight_difference, bin_weight_difference_start)
  return final.reshape(_B, _NODE), attn.reshape(_B, _ND)


# speedup vs baseline: 1.2919x; 1.0930x over previous
import functools

import numpy as np
import jax
import jax.numpy as jnp
from jax import lax
from jax.experimental import pallas as pl
from jax.experimental.pallas import tpu as pltpu
from jax.experimental.pallas import tpu_sc as plsc

_B, _S, _W, _D = 8, 16, 32, 64
_ND = _S * _W
_NV = 4096
_NODE = 32
_VT = 4096
_CH = 512
_NVT = _NV // _VT
_EDGES = [float(x) for x in np.linspace(-0.5, 0.99, 15)]


_NW = 32
_RPW = (_B * _ND) // _NW
_VOCAB = 100001
_SPLIT = 51200
_TBLK = 391


def _repack_table(table_t):
  def body(a_ref, b_ref, out_ref):
    out_ref[:, 0:_D] = jnp.transpose(a_ref[...])
    out_ref[:, _D:128] = jnp.transpose(b_ref[...])

  nblk = _SPLIT // 2048
  lastb = (_VOCAB + 2047) // 2048 - 1
  return pl.pallas_call(
      body,
      grid=(nblk,),
      in_specs=[
          pl.BlockSpec((_D, 2048), lambda c: (0, c)),
          pl.BlockSpec((_D, 2048), lambda c: (0, jnp.minimum(c + nblk, lastb))),
      ],
      out_specs=pl.BlockSpec((2048, 128), lambda c: (c, 0)),
      out_shape=jax.ShapeDtypeStruct((_SPLIT, 128), jnp.float32),
  )(table_t, table_t)


def _sc_gather(table, idx):
  mesh = plsc.VectorSubcoreMesh(core_axis_name="c", subcore_axis_name="s")

  @functools.partial(
      pl.kernel,
      mesh=mesh,
      compiler_params=pltpu.CompilerParams(use_tc_tiling_on_sc=False),
      out_type=jax.ShapeDtypeStruct((_B * _ND, 128), jnp.float32),
      scratch_types=[
          pltpu.VMEM((_RPW,), jnp.int32),
          pltpu.VMEM((_RPW,), jnp.int32),
          pltpu.VMEM((_RPW, _D), jnp.float32),
          pltpu.SemaphoreType.DMA,
      ],
  )
  def gather_kernel(table_hbm, idx_hbm, out_hbm, idx_v, idx2_v, rows_v, sem):
    wid = lax.axis_index("s") * 2 + lax.axis_index("c")
    base = wid * _RPW
    for i in range(4):
      pltpu.sync_copy(idx_hbm.at[wid * 4 + i, pl.ds(0, _W)],
                      idx_v.at[pl.ds(i * _W, _W)])
    for i in range(_RPW // 16):
      v = idx_v[pl.ds(i * 16, 16)]
      idx2_v[pl.ds(i * 16, 16)] = v + v + jnp.where(
          v < _SPLIT, 0, 1 - 2 * _SPLIT)
    pltpu.async_copy(table_hbm.at[idx2_v], rows_v, sem).wait()
    pltpu.sync_copy(rows_v, out_hbm.at[pl.ds(base, _RPW), pl.ds(0, _D)])

  return gather_kernel(table, idx)


def _tc_body(demb_ref, vvt_ref, imp_ref, phi_ref, ww_ref, ws_ref,
             diff_ref, start_ref, final_ref, attn_ref, attn_scr):
  vt = pl.program_id(1)

  @pl.when(vt == 0)
  def _():
    demb = demb_ref[:, :_D]
    wl = jnp.sum(demb * ww_ref[...], axis=1, keepdims=True)
    wl3 = wl.reshape(_S, _W, 1)
    wmax = jnp.max(wl3, axis=1, keepdims=True)
    we = jnp.exp(wl3 - wmax)
    wa = we / jnp.sum(we, axis=1, keepdims=True)
    sl = jnp.sum(imp_ref[...] * ws_ref[...], axis=1, keepdims=True)
    smax = jnp.max(sl, axis=0, keepdims=True)
    se = jnp.exp(sl - smax)
    sa = se / jnp.sum(se, axis=0, keepdims=True)
    attn = (wa * sa.reshape(_S, 1, 1)).reshape(_ND, 1)
    attn_scr[...] = attn
    attn_ref[...] = attn

  bw = [start_ref[0] + jnp.maximum(diff_ref[0], 0.0)]
  for k in range(1, 16):
    bw.append(bw[-1] + jnp.maximum(diff_ref[k], 0.0))
  attn = attn_scr[...]
  part = None
  for c in range(_VT // _CH):
    sim = jnp.dot(demb_ref[:, :_D], vvt_ref[:, c * _CH:(c + 1) * _CH],
                  preferred_element_type=jnp.float32)
    b3 = sim >= _EDGES[7]
    b2 = sim >= jnp.where(b3, _EDGES[11], _EDGES[3])
    e1 = jnp.where(b3, jnp.where(b2, _EDGES[13], _EDGES[9]),
                   jnp.where(b2, _EDGES[5], _EDGES[1]))
    b1 = sim >= e1
    e0hi = jnp.where(b3, jnp.where(b2, _EDGES[14], _EDGES[10]),
                     jnp.where(b2, _EDGES[6], _EDGES[2]))
    e0lo = jnp.where(b3, jnp.where(b2, _EDGES[12], _EDGES[8]),
                     jnp.where(b2, _EDGES[4], _EDGES[0]))
    b0 = sim >= jnp.where(b1, e0hi, e0lo)
    t = [jnp.where(b0, bw[2 * i + 1], bw[2 * i]) for i in range(8)]
    u = [jnp.where(b1, t[2 * j + 1], t[2 * j]) for j in range(4)]
    p = [jnp.where(b2, u[1], u[0]), jnp.where(b2, u[3], u[2])]
    w = jnp.where(b3, p[1], p[0])
    weighted = lax.dot_general(attn, w,
                               (((0,), (0,)), ((), ())),
                               preferred_element_type=jnp.float32)
    pc = lax.dot_general(weighted, phi_ref[:, c * _CH:(c + 1) * _CH],
                         (((1,), (1,)), ((), ())),
                         preferred_element_type=jnp.float32)
    part = pc if part is None else part + pc

  @pl.when(vt == 0)
  def _():
    final_ref[...] = part

  @pl.when(vt != 0)
  def _():
    final_ref[...] = final_ref[...] + part


def _tc_call(demb3, VvT, imp, phi, ww2, ws2, diff, start):
  return pl.pallas_call(
      _tc_body,
      grid=(_B, _NVT),
      in_specs=[
          pl.BlockSpec((_ND, 128), lambda b, v: (b, 0)),
          pl.BlockSpec((_D, _VT), lambda b, v: (0, v)),
          pl.BlockSpec((None, _S, _D), lambda b, v: (b, 0, 0)),
          pl.BlockSpec((_NODE, _VT), lambda b, v: (0, v)),
          pl.BlockSpec((1, _D), lambda b, v: (0, 0)),
          pl.BlockSpec((1, _D), lambda b, v: (0, 0)),
          pl.BlockSpec(memory_space=pltpu.SMEM),
          pl.BlockSpec(memory_space=pltpu.SMEM),
      ],
      out_specs=[
          pl.BlockSpec((None, 1, _NODE), lambda b, v: (b, 0, 0)),
          pl.BlockSpec((None, _ND, 1), lambda b, v: (b, 0, 0)),
      ],
      out_shape=[
          jax.ShapeDtypeStruct((_B, 1, _NODE), jnp.float32),
          jax.ShapeDtypeStruct((_B, _ND, 1), jnp.float32),
      ],
      scratch_shapes=[pltpu.VMEM((_ND, 1), jnp.float32)],
  )(demb3, VvT, imp, phi, ww2, ws2, diff, start)


def kernel(input_ids, ImportanceFeatureMat, labels, embedding_table, VvT,
           phi_vs, bin_weight_difference, bin_weight_difference_start,
           w_word, w_sent):
  ids = jnp.pad(input_ids.astype(jnp.int32),
                ((0, 0), (0, 0), (0, 128 - _W))).reshape(_B * _S, 128)
  tbl = _repack_table(embedding_table.T).reshape(2 * _SPLIT, _D)
  demb = _sc_gather(tbl, ids)
  final, attn = _tc_call(
      demb, VvT, ImportanceFeatureMat, phi_vs,
      w_word.reshape(1, _D), w_sent.reshape(1, _D),
      bin_weight_difference, bin_weight_difference_start)
  return final.reshape(_B, _NODE), attn.reshape(_B, _ND)


# speedup vs baseline: 1.3022x; 1.0080x over previous
import functools

import numpy as np
import jax
import jax.numpy as jnp
from jax import lax
from jax.experimental import pallas as pl
from jax.experimental.pallas import tpu as pltpu
from jax.experimental.pallas import tpu_sc as plsc

_B, _S, _W, _D = 8, 16, 32, 64
_ND = _S * _W
_NV = 4096
_NODE = 32
_VT = 4096
_CH = 512
_NVT = _NV // _VT
_EDGES = [float(x) for x in np.linspace(-0.5, 0.99, 15)]


_NW = 32
_RPW = (_B * _ND) // _NW
_VOCAB = 100001
_SPLIT = 51200
_TBLK = 391


def _repack_table(table_t):
  def body(a_ref, b_ref, out_ref):
    out_ref[:, 0:_D] = jnp.transpose(a_ref[...])
    out_ref[:, _D:128] = jnp.transpose(b_ref[...])

  nblk = _SPLIT // 2048
  lastb = (_VOCAB + 2047) // 2048 - 1
  return pl.pallas_call(
      body,
      grid=(nblk,),
      in_specs=[
          pl.BlockSpec((_D, 2048), lambda c: (0, c)),
          pl.BlockSpec((_D, 2048), lambda c: (0, jnp.minimum(c + nblk, lastb))),
      ],
      out_specs=pl.BlockSpec((2048, 128), lambda c: (c, 0)),
      out_shape=jax.ShapeDtypeStruct((_SPLIT, 128), jnp.float32),
  )(table_t, table_t)


def _sc_gather(table, idx):
  mesh = plsc.VectorSubcoreMesh(core_axis_name="c", subcore_axis_name="s")

  @functools.partial(
      pl.kernel,
      mesh=mesh,
      compiler_params=pltpu.CompilerParams(use_tc_tiling_on_sc=False),
      out_type=jax.ShapeDtypeStruct((_B * _ND, 128), jnp.float32),
      scratch_types=[
          pltpu.VMEM((_RPW,), jnp.int32),
          pltpu.VMEM((_RPW,), jnp.int32),
          pltpu.VMEM((_RPW, _D), jnp.float32),
          pltpu.SemaphoreType.DMA,
      ],
  )
  def gather_kernel(table_hbm, idx_hbm, out_hbm, idx_v, idx2_v, rows_v, sem):
    wid = lax.axis_index("s") * 2 + lax.axis_index("c")
    base = wid * _RPW
    copies = [pltpu.async_copy(idx_hbm.at[wid * 4 + i, pl.ds(0, _W)],
                               idx_v.at[pl.ds(i * _W, _W)], sem)
              for i in range(4)]
    for c in copies:
      c.wait()
    for i in range(_RPW // 16):
      v = idx_v[pl.ds(i * 16, 16)]
      idx2_v[pl.ds(i * 16, 16)] = v + v + jnp.where(
          v < _SPLIT, 0, 1 - 2 * _SPLIT)
    pltpu.async_copy(table_hbm.at[idx2_v], rows_v, sem).wait()
    pltpu.sync_copy(rows_v, out_hbm.at[pl.ds(base, _RPW), pl.ds(0, _D)])

  return gather_kernel(table, idx)


def _tc_body(demb_ref, vvt_ref, imp_ref, phi_ref, ww_ref, ws_ref,
             diff_ref, start_ref, final_ref, attn_ref, attn_scr):
  vt = pl.program_id(1)

  @pl.when(vt == 0)
  def _():
    demb = demb_ref[:, :_D]
    wl = jnp.sum(demb * ww_ref[...], axis=1, keepdims=True)
    wl3 = wl.reshape(_S, _W, 1)
    wmax = jnp.max(wl3, axis=1, keepdims=True)
    we = jnp.exp(wl3 - wmax)
    wa = we / jnp.sum(we, axis=1, keepdims=True)
    sl = jnp.sum(imp_ref[...] * ws_ref[...], axis=1, keepdims=True)
    smax = jnp.max(sl, axis=0, keepdims=True)
    se = jnp.exp(sl - smax)
    sa = se / jnp.sum(se, axis=0, keepdims=True)
    attn = (wa * sa.reshape(_S, 1, 1)).reshape(_ND, 1)
    attn_scr[...] = attn
    attn_ref[...] = attn

  bw = [start_ref[0] + jnp.maximum(diff_ref[0], 0.0)]
  for k in range(1, 16):
    bw.append(bw[-1] + jnp.maximum(diff_ref[k], 0.0))
  attn = attn_scr[...]
  part = None
  for c in range(_VT // _CH):
    sim = jnp.dot(demb_ref[:, :_D], vvt_ref[:, c * _CH:(c + 1) * _CH],
                  preferred_element_type=jnp.float32)
    b3 = sim >= _EDGES[7]
    b2 = sim >= jnp.where(b3, _EDGES[11], _EDGES[3])
    e1 = jnp.where(b3, jnp.where(b2, _EDGES[13], _EDGES[9]),
                   jnp.where(b2, _EDGES[5], _EDGES[1]))
    b1 = sim >= e1
    e0hi = jnp.where(b3, jnp.where(b2, _EDGES[14], _EDGES[10]),
                     jnp.where(b2, _EDGES[6], _EDGES[2]))
    e0lo = jnp.where(b3, jnp.where(b2, _EDGES[12], _EDGES[8]),
                     jnp.where(b2, _EDGES[4], _EDGES[0]))
    b0 = sim >= jnp.where(b1, e0hi, e0lo)
    t = [jnp.where(b0, bw[2 * i + 1], bw[2 * i]) for i in range(8)]
    u = [jnp.where(b1, t[2 * j + 1], t[2 * j]) for j in range(4)]
    p = [jnp.where(b2, u[1], u[0]), jnp.where(b2, u[3], u[2])]
    w = jnp.where(b3, p[1], p[0])
    weighted = lax.dot_general(attn, w,
                               (((0,), (0,)), ((), ())),
                               preferred_element_type=jnp.float32)
    pc = lax.dot_general(weighted, phi_ref[:, c * _CH:(c + 1) * _CH],
                         (((1,), (1,)), ((), ())),
                         preferred_element_type=jnp.float32)
    part = pc if part is None else part + pc

  @pl.when(vt == 0)
  def _():
    final_ref[...] = part

  @pl.when(vt != 0)
  def _():
    final_ref[...] = final_ref[...] + part


def _tc_call(demb3, VvT, imp, phi, ww2, ws2, diff, start):
  return pl.pallas_call(
      _tc_body,
      grid=(_B, _NVT),
      in_specs=[
          pl.BlockSpec((_ND, 128), lambda b, v: (b, 0)),
          pl.BlockSpec((_D, _VT), lambda b, v: (0, v)),
          pl.BlockSpec((None, _S, _D), lambda b, v: (b, 0, 0)),
          pl.BlockSpec((_NODE, _VT), lambda b, v: (0, v)),
          pl.BlockSpec((1, _D), lambda b, v: (0, 0)),
          pl.BlockSpec((1, _D), lambda b, v: (0, 0)),
          pl.BlockSpec(memory_space=pltpu.SMEM),
          pl.BlockSpec(memory_space=pltpu.SMEM),
      ],
      out_specs=[
          pl.BlockSpec((None, 1, _NODE), lambda b, v: (b, 0, 0)),
          pl.BlockSpec((None, _ND, 1), lambda b, v: (b, 0, 0)),
      ],
      out_shape=[
          jax.ShapeDtypeStruct((_B, 1, _NODE), jnp.float32),
          jax.ShapeDtypeStruct((_B, _ND, 1), jnp.float32),
      ],
      scratch_shapes=[pltpu.VMEM((_ND, 1), jnp.float32)],
      compiler_params=pltpu.CompilerParams(
          dimension_semantics=("parallel", "arbitrary")),
  )(demb3, VvT, imp, phi, ww2, ws2, diff, start)


def kernel(input_ids, ImportanceFeatureMat, labels, embedding_table, VvT,
           phi_vs, bin_weight_difference, bin_weight_difference_start,
           w_word, w_sent):
  ids = jnp.pad(input_ids.astype(jnp.int32),
                ((0, 0), (0, 0), (0, 128 - _W))).reshape(_B * _S, 128)
  tbl = _repack_table(embedding_table.T).reshape(2 * _SPLIT, _D)
  demb = _sc_gather(tbl, ids)
  final, attn = _tc_call(
      demb, VvT, ImportanceFeatureMat, phi_vs,
      w_word.reshape(1, _D), w_sent.reshape(1, _D),
      bin_weight_difference, bin_weight_difference_start)
  return final.reshape(_B, _NODE), attn.reshape(_B, _ND)
